# baseline (speedup 1.0000x reference)
import functools

import jax
import jax.numpy as jnp
from jax.experimental import pallas as pl
from jax.experimental.pallas import tpu as pltpu

_SEED = 64
_KBLK = 1
_PCH = 8


def _dup_even_odd(row, even, d):
    sb = jnp.where(even, row, pltpu.roll(row, 1, axis=1))
    cb = jnp.where(even, pltpu.roll(row, d - 1, axis=1), row)
    return sb, cb


def _gen_body(pe_ref, o_ref, vS, vseedS, vseedC, vSB, vCB, vrot, rsems, wsems, *, seq, d):
    nrot = (seq // _SEED).bit_length() - 1
    pltpu.make_async_copy(
        pe_ref.at[pl.ds(0, _SEED)], vseedS.at[pl.ds(0, _SEED)], rsems.at[nrot]
    ).start()
    for j in range(nrot):
        pltpu.make_async_copy(
            pe_ref.at[pl.ds(_SEED << j, 1)], vrot.at[pl.ds(j, 1)], rsems.at[j]
        ).start()

    col = jax.lax.broadcasted_iota(jnp.int32, (1, d), 1)
    even1 = (col & 1) == 0

    vSB[pl.ds(0, 1)] = jnp.zeros((1, d), jnp.float32)
    vCB[pl.ds(0, 1)] = jnp.ones((1, d), jnp.float32)

    def _rot_step(j):
        B = 1 << j
        pltpu.make_async_copy(
            pe_ref.at[pl.ds(_SEED << j, 1)], vrot.at[pl.ds(j, 1)], rsems.at[j]
        ).wait()
        sbR, cbR = _dup_even_odd(vrot[pl.ds(j, 1)], even1, d)
        sb_src = vSB[pl.ds(0, B)]
        cb_src = vCB[pl.ds(0, B)]
        vSB[pl.ds(B, B)] = sb_src * cbR + cb_src * sbR
        vCB[pl.ds(B, B)] = cb_src * cbR - sb_src * sbR

    _rot_step(0)

    pltpu.make_async_copy(
        pe_ref.at[pl.ds(0, _SEED)], vseedS.at[pl.ds(0, _SEED)], rsems.at[nrot]
    ).wait()
    evenS = (jax.lax.broadcasted_iota(jnp.int32, (_SEED, d), 1) & 1) == 0
    s_all = vseedS[pl.ds(0, _SEED)]
    vseedC[pl.ds(0, _SEED)] = jnp.where(
        evenS, pltpu.roll(s_all, d - 1, axis=1), -pltpu.roll(s_all, 1, axis=1)
    )

    bursts = [(0, 1), (1, 1), (2, 2), (4, 4), (8, 8), (16, 16), (32, 16), (48, 16)]
    rot_before = {1 << j: j for j in range(1, nrot)}
    for bi, (k0, kn) in enumerate(bursts):
        if k0 in rot_before:
            _rot_step(rot_before[k0])
        for pc in range(_SEED // _PCH):
            sS = vseedS[pl.ds(pc * _PCH, _PCH)]
            sC = vseedC[pl.ds(pc * _PCH, _PCH)]
            for k in range(k0, k0 + kn):
                cb = vCB[pl.ds(k, 1)]
                sb = vSB[pl.ds(k, 1)]
                vS[pl.ds(k * _SEED + pc * _PCH, _PCH)] = sS * cb + sC * sb
        pltpu.make_async_copy(
            vS.at[pl.ds(k0 * _SEED, kn * _SEED)],
            o_ref.at[pl.ds(k0 * _SEED, kn * _SEED)],
            wsems.at[bi],
        ).start()

    for bi, (k0, kn) in enumerate(bursts):
        pltpu.make_async_copy(
            vS.at[pl.ds(k0 * _SEED, kn * _SEED)],
            o_ref.at[pl.ds(k0 * _SEED, kn * _SEED)],
            wsems.at[bi],
        ).wait()


def kernel(x, pos_emb):
    seq = x.shape[1]
    d = pos_emb.shape[1]
    nrot = (seq // _SEED).bit_length() - 1
    nkb = 8
    body = functools.partial(_gen_body, seq=seq, d=d)
    out = pl.pallas_call(
        body,
        in_specs=[pl.BlockSpec(memory_space=pl.ANY)],
        out_specs=pl.BlockSpec(memory_space=pl.ANY),
        out_shape=jax.ShapeDtypeStruct((seq, d), jnp.float32),
        scratch_shapes=[
            pltpu.VMEM((seq, d), jnp.float32),
            pltpu.VMEM((_SEED, d), jnp.float32),
            pltpu.VMEM((_SEED, d), jnp.float32),
            pltpu.VMEM((_SEED, d), jnp.float32),
            pltpu.VMEM((_SEED, d), jnp.float32),
            pltpu.VMEM((nrot, d), jnp.float32),
            pltpu.SemaphoreType.DMA((nrot + 1,)),
            pltpu.SemaphoreType.DMA((nkb,)),
        ],
    )(pos_emb)
    return out


# speedup vs baseline: 1.0858x; 1.0858x over previous
import functools

import jax
import jax.numpy as jnp
from jax.experimental import pallas as pl
from jax.experimental.pallas import tpu as pltpu

_SEED = 64
_KBLK = 2
_PCH = 8


def _dup_even_odd(row, even, d):
    sb = jnp.where(even, row, pltpu.roll(row, 1, axis=1))
    cb = jnp.where(even, pltpu.roll(row, d - 1, axis=1), row)
    return sb, cb


def _gen_body(pe_ref, o_ref, vS, vseedS, vseedC, vSB, vCB, vrot, rsems, wsems, *, seq, d):
    nrot = (seq // _SEED).bit_length() - 1
    pltpu.make_async_copy(
        pe_ref.at[pl.ds(0, _SEED)], vseedS.at[pl.ds(0, _SEED)], rsems.at[nrot]
    ).start()
    for j in range(nrot):
        pltpu.make_async_copy(
            pe_ref.at[pl.ds(_SEED << j, 1)], vrot.at[pl.ds(j, 1)], rsems.at[j]
        ).start()

    col = jax.lax.broadcasted_iota(jnp.int32, (1, d), 1)
    even1 = (col & 1) == 0

    vSB[pl.ds(0, 1)] = jnp.zeros((1, d), jnp.float32)
    vCB[pl.ds(0, 1)] = jnp.ones((1, d), jnp.float32)

    def _rot_step(j):
        B = 1 << j
        pltpu.make_async_copy(
            pe_ref.at[pl.ds(_SEED << j, 1)], vrot.at[pl.ds(j, 1)], rsems.at[j]
        ).wait()
        sbR, cbR = _dup_even_odd(vrot[pl.ds(j, 1)], even1, d)
        sb_src = vSB[pl.ds(0, B)]
        cb_src = vCB[pl.ds(0, B)]
        vSB[pl.ds(B, B)] = sb_src * cbR + cb_src * sbR
        vCB[pl.ds(B, B)] = cb_src * cbR - sb_src * sbR

    nkb = seq // (_SEED * _KBLK)
    kblk_per_rot = {}
    eager = max(_KBLK.bit_length() - 1, 0)
    for j in range(eager, nrot):
        kblk_per_rot[(1 << j) // _KBLK] = j
    for j in range(eager):
        _rot_step(j)

    pltpu.make_async_copy(
        pe_ref.at[pl.ds(0, _SEED)], vseedS.at[pl.ds(0, _SEED)], rsems.at[nrot]
    ).wait()
    evenS = (jax.lax.broadcasted_iota(jnp.int32, (_SEED, d), 1) & 1) == 0
    s_all = vseedS[pl.ds(0, _SEED)]
    vseedC[pl.ds(0, _SEED)] = jnp.where(
        evenS, pltpu.roll(s_all, d - 1, axis=1), -pltpu.roll(s_all, 1, axis=1)
    )

    for kb in range(nkb):
        if kb in kblk_per_rot:
            _rot_step(kblk_per_rot[kb])
        for pc in range(_SEED // _PCH):
            sS = vseedS[pl.ds(pc * _PCH, _PCH)]
            sC = vseedC[pl.ds(pc * _PCH, _PCH)]
            for k in range(kb * _KBLK, (kb + 1) * _KBLK):
                cb = vCB[pl.ds(k, 1)]
                sb = vSB[pl.ds(k, 1)]
                vS[pl.ds(k * _SEED + pc * _PCH, _PCH)] = sS * cb + sC * sb
        rows = _SEED * _KBLK
        pltpu.make_async_copy(
            vS.at[pl.ds(kb * rows, rows)],
            o_ref.at[pl.ds(kb * rows, rows)],
            wsems.at[kb],
        ).start()

    rows = _SEED * _KBLK
    for kb in range(nkb):
        pltpu.make_async_copy(
            vS.at[pl.ds(kb * rows, rows)],
            o_ref.at[pl.ds(kb * rows, rows)],
            wsems.at[kb],
        ).wait()


def kernel(x, pos_emb):
    seq = x.shape[1]
    d = pos_emb.shape[1]
    nrot = (seq // _SEED).bit_length() - 1
    nkb = seq // (_SEED * _KBLK)
    body = functools.partial(_gen_body, seq=seq, d=d)
    out = pl.pallas_call(
        body,
        in_specs=[pl.BlockSpec(memory_space=pl.ANY)],
        out_specs=pl.BlockSpec(memory_space=pl.ANY),
        out_shape=jax.ShapeDtypeStruct((seq, d), jnp.float32),
        scratch_shapes=[
            pltpu.VMEM((seq, d), jnp.float32),
            pltpu.VMEM((_SEED, d), jnp.float32),
            pltpu.VMEM((_SEED, d), jnp.float32),
            pltpu.VMEM((_SEED, d), jnp.float32),
            pltpu.VMEM((_SEED, d), jnp.float32),
            pltpu.VMEM((nrot, d), jnp.float32),
            pltpu.SemaphoreType.DMA((nrot + 1,)),
            pltpu.SemaphoreType.DMA((nkb,)),
        ],
    )(pos_emb)
    return out


# speedup vs baseline: 1.0905x; 1.0044x over previous
import functools

import jax
import jax.numpy as jnp
from jax.experimental import pallas as pl
from jax.experimental.pallas import tpu as pltpu

_SEED = 64
_KBLK = 2
_PCH = 16


def _dup_even_odd(row, even, d):
    sb = jnp.where(even, row, pltpu.roll(row, 1, axis=1))
    cb = jnp.where(even, pltpu.roll(row, d - 1, axis=1), row)
    return sb, cb


def _gen_body(pe_ref, o_ref, vS, vseedS, vseedC, vSB, vCB, vrot, rsems, wsems, *, seq, d):
    nrot = (seq // _SEED).bit_length() - 1
    pltpu.make_async_copy(
        pe_ref.at[pl.ds(0, _SEED)], vseedS.at[pl.ds(0, _SEED)], rsems.at[nrot]
    ).start()
    for j in range(nrot):
        pltpu.make_async_copy(
            pe_ref.at[pl.ds(_SEED << j, 1)], vrot.at[pl.ds(j, 1)], rsems.at[j]
        ).start()

    col = jax.lax.broadcasted_iota(jnp.int32, (1, d), 1)
    even1 = (col & 1) == 0

    vSB[pl.ds(0, 1)] = jnp.zeros((1, d), jnp.float32)
    vCB[pl.ds(0, 1)] = jnp.ones((1, d), jnp.float32)

    def _rot_step(j):
        B = 1 << j
        pltpu.make_async_copy(
            pe_ref.at[pl.ds(_SEED << j, 1)], vrot.at[pl.ds(j, 1)], rsems.at[j]
        ).wait()
        sbR, cbR = _dup_even_odd(vrot[pl.ds(j, 1)], even1, d)
        sb_src = vSB[pl.ds(0, B)]
        cb_src = vCB[pl.ds(0, B)]
        vSB[pl.ds(B, B)] = sb_src * cbR + cb_src * sbR
        vCB[pl.ds(B, B)] = cb_src * cbR - sb_src * sbR

    nkb = seq // (_SEED * _KBLK)
    kblk_per_rot = {}
    eager = max(_KBLK.bit_length() - 1, 0)
    for j in range(eager, nrot):
        kblk_per_rot[(1 << j) // _KBLK] = j
    for j in range(eager):
        _rot_step(j)

    pltpu.make_async_copy(
        pe_ref.at[pl.ds(0, _SEED)], vseedS.at[pl.ds(0, _SEED)], rsems.at[nrot]
    ).wait()
    evenS = (jax.lax.broadcasted_iota(jnp.int32, (_SEED, d), 1) & 1) == 0
    s_all = vseedS[pl.ds(0, _SEED)]
    vseedC[pl.ds(0, _SEED)] = jnp.where(
        evenS, pltpu.roll(s_all, d - 1, axis=1), -pltpu.roll(s_all, 1, axis=1)
    )

    for kb in range(nkb):
        if kb in kblk_per_rot:
            _rot_step(kblk_per_rot[kb])
        for pc in range(_SEED // _PCH):
            sS = vseedS[pl.ds(pc * _PCH, _PCH)]
            sC = vseedC[pl.ds(pc * _PCH, _PCH)]
            for k in range(kb * _KBLK, (kb + 1) * _KBLK):
                cb = vCB[pl.ds(k, 1)]
                sb = vSB[pl.ds(k, 1)]
                vS[pl.ds(k * _SEED + pc * _PCH, _PCH)] = sS * cb + sC * sb
        rows = _SEED * _KBLK
        pltpu.make_async_copy(
            vS.at[pl.ds(kb * rows, rows)],
            o_ref.at[pl.ds(kb * rows, rows)],
            wsems.at[kb],
        ).start()

    rows = _SEED * _KBLK
    for kb in range(nkb):
        pltpu.make_async_copy(
            vS.at[pl.ds(kb * rows, rows)],
            o_ref.at[pl.ds(kb * rows, rows)],
            wsems.at[kb],
        ).wait()


def kernel(x, pos_emb):
    seq = x.shape[1]
    d = pos_emb.shape[1]
    nrot = (seq // _SEED).bit_length() - 1
    nkb = seq // (_SEED * _KBLK)
    body = functools.partial(_gen_body, seq=seq, d=d)
    out = pl.pallas_call(
        body,
        in_specs=[pl.BlockSpec(memory_space=pl.ANY)],
        out_specs=pl.BlockSpec(memory_space=pl.ANY),
        out_shape=jax.ShapeDtypeStruct((seq, d), jnp.float32),
        scratch_shapes=[
            pltpu.VMEM((seq, d), jnp.float32),
            pltpu.VMEM((_SEED, d), jnp.float32),
            pltpu.VMEM((_SEED, d), jnp.float32),
            pltpu.VMEM((_SEED, d), jnp.float32),
            pltpu.VMEM((_SEED, d), jnp.float32),
            pltpu.VMEM((nrot, d), jnp.float32),
            pltpu.SemaphoreType.DMA((nrot + 1,)),
            pltpu.SemaphoreType.DMA((nkb,)),
        ],
    )(pos_emb)
    return out
